# baseline (speedup 1.0000x reference)
import functools

import jax
import jax.numpy as jnp
from jax import lax
from jax.experimental import pallas as pl
from jax.experimental.pallas import tpu as pltpu
from jax.experimental.pallas import tpu_sc as plsc

SDF_MIN = -7.0
SDF_MAX = 7.0
N_BINS = 256

_NC = 2
_NS = 16
_NW = _NC * _NS
_LANES = 16
_CHUNK = 16384


def _wse_partials(y_pred_flat, y_true_flat, lut, n_sc):
    n = y_pred_flat.shape[0]
    per_w = n_sc // _NW
    assert per_w % _CHUNK == 0 and (per_w // _CHUNK) % 2 == 0
    n_chunks = per_w // _CHUNK
    mesh = plsc.VectorSubcoreMesh(core_axis_name="c", subcore_axis_name="s")

    @functools.partial(
        pl.kernel,
        mesh=mesh,
        out_type=jax.ShapeDtypeStruct((_NW, _LANES), jnp.float32),
        scratch_types=[
            pltpu.VMEM((N_BINS,), jnp.float32),
            pltpu.VMEM((_CHUNK,), jnp.float32),
            pltpu.VMEM((_CHUNK,), jnp.float32),
            pltpu.VMEM((_CHUNK,), jnp.float32),
            pltpu.VMEM((_CHUNK,), jnp.float32),
            pltpu.VMEM((_LANES,), jnp.float32),
            pltpu.SemaphoreType.DMA,
            pltpu.SemaphoreType.DMA,
            pltpu.SemaphoreType.DMA,
            pltpu.SemaphoreType.DMA,
        ],
        compiler_params=pltpu.CompilerParams(needs_layout_passes=False),
    )
    def k(pred_hbm, true_hbm, lut_hbm, out_hbm,
          lut_v, pred0, true0, pred1, true1, acc_v, sp0, st0, sp1, st1):
        wid = lax.axis_index("s") * _NC + lax.axis_index("c")
        base = wid * per_w
        pltpu.sync_copy(lut_hbm, lut_v)

        off_cap = n - _CHUNK
        scale = 1.0 / (SDF_MAX - SDF_MIN)

        def issue(pred_v, true_v, sp, st, off):
            pltpu.async_copy(pred_hbm.at[pl.ds(off, _CHUNK)], pred_v, sp)
            pltpu.async_copy(true_hbm.at[pl.ds(off, _CHUNK)], true_v, st)

        def wait(pred_v, true_v, sp, st):
            pltpu.make_async_copy(pred_hbm.at[pl.ds(0, _CHUNK)], pred_v, sp).wait()
            pltpu.make_async_copy(true_hbm.at[pl.ds(0, _CHUNK)], true_v, st).wait()

        bin_a = float(N_BINS - 1) * scale
        bin_b = -SDF_MIN * float(N_BINS - 1) * scale + 0.5

        def compute(pred_v, true_v, accs):
            def one(v, acc):
                t = true_v[pl.ds(v, _LANES)]
                p = pred_v[pl.ds(v, _LANES)]
                c = jnp.minimum(jnp.maximum(t, SDF_MIN), SDF_MAX)
                idx = (c * bin_a + bin_b).astype(jnp.int32)
                w = plsc.load_gather(lut_v, [idx])
                d = p - t
                return acc + w * d * d

            @plsc.parallel_loop(0, _CHUNK, step=4 * _LANES, unroll=6,
                                carry=accs)
            def body(v, accs):
                return tuple(one(v + j * _LANES, a)
                             for j, a in enumerate(accs))

            return body

        issue(pred0, true0, sp0, st0, base)

        def body(i, accs):
            g0 = 2 * i
            issue(pred1, true1, sp1, st1, base + (g0 + 1) * _CHUNK)
            wait(pred0, true0, sp0, st0)
            accs = compute(pred0, true0, accs)
            off2 = jnp.minimum(base + (g0 + 2) * _CHUNK, off_cap)
            issue(pred0, true0, sp0, st0, off2)
            wait(pred1, true1, sp1, st1)
            accs = compute(pred1, true1, accs)
            return accs

        z = jnp.zeros((_LANES,), jnp.float32)
        accs = lax.fori_loop(0, n_chunks // 2, body, (z,) * 4)
        wait(pred0, true0, sp0, st0)
        a0, a1, a2, a3 = accs
        acc_v[...] = (a0 + a1) + (a2 + a3)
        pltpu.sync_copy(acc_v, out_hbm.at[wid])

    return k(y_pred_flat, y_true_flat, lut)


_TC_BLOCK = 4096


def _wse_tc(yp2d, yt2d, lut2, row0):
    rows = yp2d.shape[0] - row0
    assert rows % _TC_BLOCK == 0 and row0 % _TC_BLOCK == 0
    grid = (rows // _TC_BLOCK,)
    scale = 1.0 / (SDF_MAX - SDF_MIN)
    bin_a = float(N_BINS - 1) * scale
    bin_b = -SDF_MIN * float(N_BINS - 1) * scale + 0.5
    blk0 = row0 // _TC_BLOCK

    def body(p_ref, t_ref, l_ref, o_ref):
        i = pl.program_id(0)
        t = t_ref[...]
        p = p_ref[...]
        c = jnp.minimum(jnp.maximum(t, SDF_MIN), SDF_MAX)
        idx = (c * bin_a + bin_b).astype(jnp.int32)
        hi = idx >= 128
        idxm = jnp.where(hi, idx - 128, idx)
        lo_tab = jnp.broadcast_to(l_ref[0:1, :], idx.shape)
        hi_tab = jnp.broadcast_to(l_ref[1:2, :], idx.shape)
        w = jnp.where(hi,
                      jnp.take_along_axis(hi_tab, idxm, axis=1),
                      jnp.take_along_axis(lo_tab, idxm, axis=1))
        d = p - t
        s = jnp.sum(w * d * d, axis=0, keepdims=True)

        @pl.when(i == 0)
        def _():
            o_ref[...] = jnp.zeros_like(o_ref)

        o_ref[...] += s

    out = pl.pallas_call(
        body,
        grid=grid,
        in_specs=[
            pl.BlockSpec((_TC_BLOCK, 128), lambda i: (blk0 + i, 0)),
            pl.BlockSpec((_TC_BLOCK, 128), lambda i: (blk0 + i, 0)),
            pl.BlockSpec((2, 128), lambda i: (0, 0)),
        ],
        out_specs=pl.BlockSpec((1, 128), lambda i: (0, 0)),
        out_shape=jax.ShapeDtypeStruct((1, 128), jnp.float32),
    )(yp2d, yt2d, lut2)
    return out.sum()


_N_SC = 9 * 1024 * 1024


def kernel(y_pred, y_true, lut):
    n = y_pred.size
    yp = y_pred.reshape(-1)
    yt = y_true.reshape(-1)
    partials = _wse_partials(yp, yt, lut, _N_SC)
    tc_sum = _wse_tc(y_pred.reshape(-1, 128), y_true.reshape(-1, 128),
                     lut.reshape(2, 128), _N_SC // 128)
    return (partials.sum() + tc_sum) / n


# speedup vs baseline: 1.0360x; 1.0360x over previous
import functools

import jax
import jax.numpy as jnp
from jax import lax
from jax.experimental import pallas as pl
from jax.experimental.pallas import tpu as pltpu
from jax.experimental.pallas import tpu_sc as plsc

SDF_MIN = -7.0
SDF_MAX = 7.0
N_BINS = 256

_NC = 2
_NS = 16
_NW = _NC * _NS
_LANES = 16
_CHUNK = 24576


def _wse_partials(y_pred_flat, y_true_flat, lut, n_sc):
    n = y_pred_flat.shape[0]
    per_w = n_sc // _NW
    assert per_w % _CHUNK == 0 and (per_w // _CHUNK) % 2 == 0
    n_chunks = per_w // _CHUNK
    mesh = plsc.VectorSubcoreMesh(core_axis_name="c", subcore_axis_name="s")

    @functools.partial(
        pl.kernel,
        mesh=mesh,
        out_type=jax.ShapeDtypeStruct((_NW, _LANES), jnp.float32),
        scratch_types=[
            pltpu.VMEM((N_BINS,), jnp.float32),
            pltpu.VMEM((_CHUNK,), jnp.float32),
            pltpu.VMEM((_CHUNK,), jnp.float32),
            pltpu.VMEM((_CHUNK,), jnp.float32),
            pltpu.VMEM((_CHUNK,), jnp.float32),
            pltpu.VMEM((_LANES,), jnp.float32),
            pltpu.SemaphoreType.DMA,
            pltpu.SemaphoreType.DMA,
            pltpu.SemaphoreType.DMA,
            pltpu.SemaphoreType.DMA,
        ],
        compiler_params=pltpu.CompilerParams(needs_layout_passes=False),
    )
    def k(pred_hbm, true_hbm, lut_hbm, out_hbm,
          lut_v, pred0, true0, pred1, true1, acc_v, sp0, st0, sp1, st1):
        wid = lax.axis_index("s") * _NC + lax.axis_index("c")
        base = wid * per_w
        pltpu.sync_copy(lut_hbm, lut_v)

        off_cap = n - _CHUNK
        scale = 1.0 / (SDF_MAX - SDF_MIN)

        def issue(pred_v, true_v, sp, st, off):
            pltpu.async_copy(pred_hbm.at[pl.ds(off, _CHUNK)], pred_v, sp)
            pltpu.async_copy(true_hbm.at[pl.ds(off, _CHUNK)], true_v, st)

        def wait(pred_v, true_v, sp, st):
            pltpu.make_async_copy(pred_hbm.at[pl.ds(0, _CHUNK)], pred_v, sp).wait()
            pltpu.make_async_copy(true_hbm.at[pl.ds(0, _CHUNK)], true_v, st).wait()

        bin_a = float(N_BINS - 1) * scale
        bin_b = -SDF_MIN * float(N_BINS - 1) * scale + 0.5

        def compute(pred_v, true_v, accs):
            def one(v, acc):
                t = true_v[pl.ds(v, _LANES)]
                p = pred_v[pl.ds(v, _LANES)]
                c = jnp.minimum(jnp.maximum(t, SDF_MIN), SDF_MAX)
                idx = (c * bin_a + bin_b).astype(jnp.int32)
                w = plsc.load_gather(lut_v, [idx])
                d = p - t
                return acc + w * d * d

            @plsc.parallel_loop(0, _CHUNK, step=4 * _LANES, unroll=4,
                                carry=accs)
            def body(v, accs):
                return tuple(one(v + j * _LANES, a)
                             for j, a in enumerate(accs))

            return body

        issue(pred0, true0, sp0, st0, base)

        def body(i, accs):
            g0 = 2 * i
            issue(pred1, true1, sp1, st1, base + (g0 + 1) * _CHUNK)
            wait(pred0, true0, sp0, st0)
            accs = compute(pred0, true0, accs)
            off2 = jnp.minimum(base + (g0 + 2) * _CHUNK, off_cap)
            issue(pred0, true0, sp0, st0, off2)
            wait(pred1, true1, sp1, st1)
            accs = compute(pred1, true1, accs)
            return accs

        z = jnp.zeros((_LANES,), jnp.float32)
        accs = lax.fori_loop(0, n_chunks // 2, body, (z,) * 4)
        wait(pred0, true0, sp0, st0)
        a0, a1, a2, a3 = accs
        acc_v[...] = (a0 + a1) + (a2 + a3)
        pltpu.sync_copy(acc_v, out_hbm.at[wid])

    return k(y_pred_flat, y_true_flat, lut)


_TC_BLOCK = 4096


def _wse_tc(yp2d, yt2d, lut2, row0):
    rows = yp2d.shape[0] - row0
    assert rows % _TC_BLOCK == 0 and row0 % _TC_BLOCK == 0
    grid = (rows // _TC_BLOCK,)
    scale = 1.0 / (SDF_MAX - SDF_MIN)
    bin_a = float(N_BINS - 1) * scale
    bin_b = -SDF_MIN * float(N_BINS - 1) * scale + 0.5
    blk0 = row0 // _TC_BLOCK

    def body(p_ref, t_ref, l_ref, o_ref):
        i = pl.program_id(0)
        t = t_ref[...]
        p = p_ref[...]
        c = jnp.minimum(jnp.maximum(t, SDF_MIN), SDF_MAX)
        idx = (c * bin_a + bin_b).astype(jnp.int32)
        hi = idx >= 128
        idxm = jnp.where(hi, idx - 128, idx)
        lo_tab = jnp.broadcast_to(l_ref[0:1, :], idx.shape)
        hi_tab = jnp.broadcast_to(l_ref[1:2, :], idx.shape)
        w = jnp.where(hi,
                      jnp.take_along_axis(hi_tab, idxm, axis=1),
                      jnp.take_along_axis(lo_tab, idxm, axis=1))
        d = p - t
        s = jnp.sum(w * d * d, axis=0, keepdims=True)

        @pl.when(i == 0)
        def _():
            o_ref[...] = jnp.zeros_like(o_ref)

        o_ref[...] += s

    out = pl.pallas_call(
        body,
        grid=grid,
        in_specs=[
            pl.BlockSpec((_TC_BLOCK, 128), lambda i: (blk0 + i, 0)),
            pl.BlockSpec((_TC_BLOCK, 128), lambda i: (blk0 + i, 0)),
            pl.BlockSpec((2, 128), lambda i: (0, 0)),
        ],
        out_specs=pl.BlockSpec((1, 128), lambda i: (0, 0)),
        out_shape=jax.ShapeDtypeStruct((1, 128), jnp.float32),
    )(yp2d, yt2d, lut2)
    return out.sum()


_N_SC = 9 * 1024 * 1024


def kernel(y_pred, y_true, lut):
    n = y_pred.size
    yp = y_pred.reshape(-1)
    yt = y_true.reshape(-1)
    partials = _wse_partials(yp, yt, lut, _N_SC)
    tc_sum = _wse_tc(y_pred.reshape(-1, 128), y_true.reshape(-1, 128),
                     lut.reshape(2, 128), _N_SC // 128)
    return (partials.sum() + tc_sum) / n


# speedup vs baseline: 1.0683x; 1.0312x over previous
import functools

import jax
import jax.numpy as jnp
from jax import lax
from jax.experimental import pallas as pl
from jax.experimental.pallas import tpu as pltpu
from jax.experimental.pallas import tpu_sc as plsc

SDF_MIN = -7.0
SDF_MAX = 7.0
N_BINS = 256

_NC = 2
_NS = 16
_NW = _NC * _NS
_LANES = 16
_CHUNK = 16384


def _wse_partials(y_pred_flat, y_true_flat, lut, n_sc):
    n = y_pred_flat.shape[0]
    per_w = n_sc // _NW
    assert per_w % _CHUNK == 0 and (per_w // _CHUNK) % 2 == 0
    n_chunks = per_w // _CHUNK
    mesh = plsc.VectorSubcoreMesh(core_axis_name="c", subcore_axis_name="s")

    @functools.partial(
        pl.kernel,
        mesh=mesh,
        out_type=jax.ShapeDtypeStruct((_NW, _LANES), jnp.float32),
        scratch_types=[
            pltpu.VMEM((N_BINS,), jnp.float32),
            pltpu.VMEM((_CHUNK,), jnp.float32),
            pltpu.VMEM((_CHUNK,), jnp.float32),
            pltpu.VMEM((_CHUNK,), jnp.float32),
            pltpu.VMEM((_CHUNK,), jnp.float32),
            pltpu.VMEM((_LANES,), jnp.float32),
            pltpu.SemaphoreType.DMA,
            pltpu.SemaphoreType.DMA,
            pltpu.SemaphoreType.DMA,
            pltpu.SemaphoreType.DMA,
        ],
        compiler_params=pltpu.CompilerParams(needs_layout_passes=False),
    )
    def k(pred_hbm, true_hbm, lut_hbm, out_hbm,
          lut_v, pred0, true0, pred1, true1, acc_v, sp0, st0, sp1, st1):
        wid = lax.axis_index("s") * _NC + lax.axis_index("c")
        base = wid * per_w
        pltpu.sync_copy(lut_hbm, lut_v)

        off_cap = n - _CHUNK
        scale = 1.0 / (SDF_MAX - SDF_MIN)

        def issue(pred_v, true_v, sp, st, off):
            pltpu.async_copy(pred_hbm.at[pl.ds(off, _CHUNK)], pred_v, sp)
            pltpu.async_copy(true_hbm.at[pl.ds(off, _CHUNK)], true_v, st)

        def wait(pred_v, true_v, sp, st):
            pltpu.make_async_copy(pred_hbm.at[pl.ds(0, _CHUNK)], pred_v, sp).wait()
            pltpu.make_async_copy(true_hbm.at[pl.ds(0, _CHUNK)], true_v, st).wait()

        bin_a = float(N_BINS - 1) * scale
        bin_b = -SDF_MIN * float(N_BINS - 1) * scale + 0.5

        def compute(pred_v, true_v, accs):
            def one(v, acc):
                t = true_v[pl.ds(v, _LANES)]
                p = pred_v[pl.ds(v, _LANES)]
                c = jnp.minimum(jnp.maximum(t, SDF_MIN), SDF_MAX)
                idx = (c * bin_a + bin_b).astype(jnp.int32)
                w = plsc.load_gather(lut_v, [idx])
                d = p - t
                return acc + w * d * d

            @plsc.parallel_loop(0, _CHUNK, step=4 * _LANES, unroll=4,
                                carry=accs)
            def body(v, accs):
                return tuple(one(v + j * _LANES, a)
                             for j, a in enumerate(accs))

            return body

        issue(pred0, true0, sp0, st0, base)

        def body(i, accs):
            g0 = 2 * i
            issue(pred1, true1, sp1, st1, base + (g0 + 1) * _CHUNK)
            wait(pred0, true0, sp0, st0)
            accs = compute(pred0, true0, accs)
            off2 = jnp.minimum(base + (g0 + 2) * _CHUNK, off_cap)
            issue(pred0, true0, sp0, st0, off2)
            wait(pred1, true1, sp1, st1)
            accs = compute(pred1, true1, accs)
            return accs

        z = jnp.zeros((_LANES,), jnp.float32)
        accs = lax.fori_loop(0, n_chunks // 2, body, (z,) * 4)
        wait(pred0, true0, sp0, st0)
        a0, a1, a2, a3 = accs
        acc_v[...] = (a0 + a1) + (a2 + a3)
        pltpu.sync_copy(acc_v, out_hbm.at[wid])

    return k(y_pred_flat, y_true_flat, lut)


_TC_BLOCK = 8192


def _wse_tc(yp2d, yt2d, lut2, row0):
    rows = yp2d.shape[0] - row0
    assert rows % _TC_BLOCK == 0 and row0 % _TC_BLOCK == 0
    grid = (rows // _TC_BLOCK,)
    scale = 1.0 / (SDF_MAX - SDF_MIN)
    bin_a = float(N_BINS - 1) * scale
    bin_b = -SDF_MIN * float(N_BINS - 1) * scale + 0.5
    blk0 = row0 // _TC_BLOCK

    def body(p_ref, t_ref, l_ref, o_ref):
        i = pl.program_id(0)
        t = t_ref[...]
        p = p_ref[...]
        c = jnp.minimum(jnp.maximum(t, SDF_MIN), SDF_MAX)
        idx = (c * bin_a + bin_b).astype(jnp.int32)
        hi = idx >= 128
        idxm = jax.lax.bitwise_and(idx, 127)
        lo_tab = jnp.broadcast_to(l_ref[0:1, :], idx.shape)
        hi_tab = jnp.broadcast_to(l_ref[1:2, :], idx.shape)
        w = jnp.where(hi,
                      jnp.take_along_axis(hi_tab, idxm, axis=1),
                      jnp.take_along_axis(lo_tab, idxm, axis=1))
        d = p - t
        s = jnp.sum(w * d * d, axis=0, keepdims=True)

        @pl.when(i == 0)
        def _():
            o_ref[...] = jnp.zeros_like(o_ref)

        o_ref[...] += s

    out = pl.pallas_call(
        body,
        grid=grid,
        in_specs=[
            pl.BlockSpec((_TC_BLOCK, 128), lambda i: (blk0 + i, 0)),
            pl.BlockSpec((_TC_BLOCK, 128), lambda i: (blk0 + i, 0)),
            pl.BlockSpec((2, 128), lambda i: (0, 0)),
        ],
        out_specs=pl.BlockSpec((1, 128), lambda i: (0, 0)),
        out_shape=jax.ShapeDtypeStruct((1, 128), jnp.float32),
    )(yp2d, yt2d, lut2)
    return out.sum()


_N_SC = 9 * 1024 * 1024


def kernel(y_pred, y_true, lut):
    n = y_pred.size
    yp = y_pred.reshape(-1)
    yt = y_true.reshape(-1)
    partials = _wse_partials(yp, yt, lut, _N_SC)
    tc_sum = _wse_tc(y_pred.reshape(-1, 128), y_true.reshape(-1, 128),
                     lut.reshape(2, 128), _N_SC // 128)
    return (partials.sum() + tc_sum) / n


# speedup vs baseline: 1.0863x; 1.0168x over previous
import functools

import jax
import jax.numpy as jnp
from jax import lax
from jax.experimental import pallas as pl
from jax.experimental.pallas import tpu as pltpu
from jax.experimental.pallas import tpu_sc as plsc

SDF_MIN = -7.0
SDF_MAX = 7.0
N_BINS = 256

_NC = 2
_NS = 16
_NW = _NC * _NS
_LANES = 16
_CHUNK = 16384


def _wse_partials(y_pred_flat, y_true_flat, lut, n_sc):
    n = y_pred_flat.shape[0]
    per_w = n_sc // _NW
    assert per_w % _CHUNK == 0 and (per_w // _CHUNK) % 2 == 0
    n_chunks = per_w // _CHUNK
    mesh = plsc.VectorSubcoreMesh(core_axis_name="c", subcore_axis_name="s")

    @functools.partial(
        pl.kernel,
        mesh=mesh,
        out_type=jax.ShapeDtypeStruct((_NW, _LANES), jnp.float32),
        scratch_types=[
            pltpu.VMEM((N_BINS,), jnp.float32),
            pltpu.VMEM((_CHUNK,), jnp.float32),
            pltpu.VMEM((_CHUNK,), jnp.float32),
            pltpu.VMEM((_CHUNK,), jnp.float32),
            pltpu.VMEM((_CHUNK,), jnp.float32),
            pltpu.VMEM((_LANES,), jnp.float32),
            pltpu.SemaphoreType.DMA,
            pltpu.SemaphoreType.DMA,
            pltpu.SemaphoreType.DMA,
            pltpu.SemaphoreType.DMA,
        ],
        compiler_params=pltpu.CompilerParams(needs_layout_passes=False),
    )
    def k(pred_hbm, true_hbm, lut_hbm, out_hbm,
          lut_v, pred0, true0, pred1, true1, acc_v, sp0, st0, sp1, st1):
        wid = lax.axis_index("s") * _NC + lax.axis_index("c")
        base = wid * per_w
        pltpu.sync_copy(lut_hbm, lut_v)

        off_cap = n - _CHUNK
        scale = 1.0 / (SDF_MAX - SDF_MIN)

        def issue(pred_v, true_v, sp, st, off):
            pltpu.async_copy(pred_hbm.at[pl.ds(off, _CHUNK)], pred_v, sp)
            pltpu.async_copy(true_hbm.at[pl.ds(off, _CHUNK)], true_v, st)

        def wait(pred_v, true_v, sp, st):
            pltpu.make_async_copy(pred_hbm.at[pl.ds(0, _CHUNK)], pred_v, sp).wait()
            pltpu.make_async_copy(true_hbm.at[pl.ds(0, _CHUNK)], true_v, st).wait()

        bin_a = float(N_BINS - 1) * scale
        bin_b = -SDF_MIN * float(N_BINS - 1) * scale + 0.5

        def compute(pred_v, true_v, accs):
            def one(v, acc):
                t = true_v[pl.ds(v, _LANES)]
                p = pred_v[pl.ds(v, _LANES)]
                c = jnp.minimum(jnp.maximum(t, SDF_MIN), SDF_MAX)
                idx = (c * bin_a + bin_b).astype(jnp.int32)
                w = plsc.load_gather(lut_v, [idx])
                d = p - t
                return acc + w * d * d

            @plsc.parallel_loop(0, _CHUNK, step=4 * _LANES, unroll=4,
                                carry=accs)
            def body(v, accs):
                return tuple(one(v + j * _LANES, a)
                             for j, a in enumerate(accs))

            return body

        issue(pred0, true0, sp0, st0, base)

        def body(i, accs):
            g0 = 2 * i
            issue(pred1, true1, sp1, st1, base + (g0 + 1) * _CHUNK)
            wait(pred0, true0, sp0, st0)
            accs = compute(pred0, true0, accs)
            off2 = jnp.minimum(base + (g0 + 2) * _CHUNK, off_cap)
            issue(pred0, true0, sp0, st0, off2)
            wait(pred1, true1, sp1, st1)
            accs = compute(pred1, true1, accs)
            return accs

        z = jnp.zeros((_LANES,), jnp.float32)
        accs = lax.fori_loop(0, n_chunks // 2, body, (z,) * 4)
        wait(pred0, true0, sp0, st0)
        a0, a1, a2, a3 = accs
        acc_v[...] = (a0 + a1) + (a2 + a3)
        pltpu.sync_copy(acc_v, out_hbm.at[wid])

    return k(y_pred_flat, y_true_flat, lut)


_TC_BLOCK = 8192


def _wse_tc(yp2d, yt2d, lut2, row0):
    rows = yp2d.shape[0] - row0
    assert rows % _TC_BLOCK == 0 and row0 % _TC_BLOCK == 0
    grid = (rows // _TC_BLOCK,)
    scale = 1.0 / (SDF_MAX - SDF_MIN)
    bin_a = float(N_BINS - 1) * scale
    bin_b = -SDF_MIN * float(N_BINS - 1) * scale + 0.5
    blk0 = row0 // _TC_BLOCK

    def body(p_ref, t_ref, l_ref, o_ref):
        i = pl.program_id(0)
        t = t_ref[...]
        p = p_ref[...]
        c = jnp.minimum(jnp.maximum(t, SDF_MIN), SDF_MAX)
        idx = (c * bin_a + bin_b).astype(jnp.int32)
        hi = idx >= 128
        idxm = jax.lax.bitwise_and(idx, 127)
        lo_tab = jnp.broadcast_to(l_ref[0:1, :], idx.shape)
        hi_tab = jnp.broadcast_to(l_ref[1:2, :], idx.shape)
        w = jnp.where(hi,
                      jnp.take_along_axis(hi_tab, idxm, axis=1),
                      jnp.take_along_axis(lo_tab, idxm, axis=1))
        d = p - t
        s = jnp.sum(w * d * d, axis=0, keepdims=True)

        @pl.when(i == 0)
        def _():
            o_ref[...] = jnp.zeros_like(o_ref)

        o_ref[...] += s

    out = pl.pallas_call(
        body,
        grid=grid,
        in_specs=[
            pl.BlockSpec((_TC_BLOCK, 128), lambda i: (blk0 + i, 0)),
            pl.BlockSpec((_TC_BLOCK, 128), lambda i: (blk0 + i, 0)),
            pl.BlockSpec((2, 128), lambda i: (0, 0)),
        ],
        out_specs=pl.BlockSpec((1, 128), lambda i: (0, 0)),
        out_shape=jax.ShapeDtypeStruct((1, 128), jnp.float32),
    )(yp2d, yt2d, lut2)
    return out.sum()


_N_SC = 8 * 1024 * 1024


def kernel(y_pred, y_true, lut):
    n = y_pred.size
    yp = y_pred.reshape(-1)
    yt = y_true.reshape(-1)
    partials = _wse_partials(yp, yt, lut, _N_SC)
    tc_sum = _wse_tc(y_pred.reshape(-1, 128), y_true.reshape(-1, 128),
                     lut.reshape(2, 128), _N_SC // 128)
    return (partials.sum() + tc_sum) / n
